# Initial kernel scaffold; baseline (speedup 1.0000x reference)
import jax
import jax.numpy as jnp
from jax.experimental import pallas as pl


def kernel(node_feat, edge_feat, edge_index, node_W, node_b, edge_W, edge_b, W1, b1, g1, be1, W2, b2, g2, be2, pred_W, pred_b):
    raise NotImplementedError("write your pallas kernel here")


# speedup vs baseline: 2.6830x; 2.6830x over previous
import functools

import jax
import jax.numpy as jnp
from jax import lax
from jax.experimental import pallas as pl
from jax.experimental.pallas import tpu as pltpu
from jax.experimental.pallas import tpu_sc as plsc

NC = 2
NS = 16
LANES = 16
NW = NC * NS


def _linear_body(x_ref, w_ref, b_ref, o_ref):
    o_ref[...] = (
        jnp.dot(x_ref[...], w_ref[...], preferred_element_type=jnp.float32)
        + b_ref[...]
    )


def _tc_linear(x, w, b, block_rows):
    m, k = x.shape
    n = w.shape[1]
    assert m % block_rows == 0
    return pl.pallas_call(
        _linear_body,
        grid=(m // block_rows,),
        in_specs=[
            pl.BlockSpec((block_rows, k), lambda i: (i, 0)),
            pl.BlockSpec((k, n), lambda i: (0, 0)),
            pl.BlockSpec((1, n), lambda i: (0, 0)),
        ],
        out_specs=pl.BlockSpec((block_rows, n), lambda i: (i, 0)),
        out_shape=jax.ShapeDtypeStruct((m, n), jnp.float32),
    )(x, w, b.reshape(1, n))


def _mlp_body(relu_out, hn_ref, a_ref, w1_ref, b1_ref, w2_ref, b2_ref, o_ref):
    h = hn_ref[...] + a_ref[0] + a_ref[1]
    y = jnp.dot(h, w1_ref[...], preferred_element_type=jnp.float32) + b1_ref[...]
    y = jnp.maximum(y, 0.0)
    z = jnp.dot(y, w2_ref[...], preferred_element_type=jnp.float32) + b2_ref[...]
    if relu_out:
        z = jnp.maximum(z, 0.0)
    o_ref[...] = z


def _tc_mlp(hn, aggr, w1, b1, w2, b2, relu_out, block_rows):
    n, d = hn.shape
    h = w1.shape[1]
    assert n % block_rows == 0
    return pl.pallas_call(
        functools.partial(_mlp_body, relu_out),
        grid=(n // block_rows,),
        in_specs=[
            pl.BlockSpec((block_rows, d), lambda i: (i, 0)),
            pl.BlockSpec((NC, block_rows, d), lambda i: (0, i, 0)),
            pl.BlockSpec((d, h), lambda i: (0, 0)),
            pl.BlockSpec((1, h), lambda i: (0, 0)),
            pl.BlockSpec((h, d), lambda i: (0, 0)),
            pl.BlockSpec((1, d), lambda i: (0, 0)),
        ],
        out_specs=pl.BlockSpec((block_rows, d), lambda i: (i, 0)),
        out_shape=jax.ShapeDtypeStruct((n, d), jnp.float32),
    )(hn, aggr, w1, b1.reshape(1, h), w2, b2.reshape(1, d))


def _pool_body(n_rows, hn_ref, wt_ref, b_ref, o_ref):
    s = jnp.sum(hn_ref[...], axis=0, keepdims=True) * (1.0 / n_rows)
    o_ref[...] = jnp.sum(s * wt_ref[...], axis=1, keepdims=True) + b_ref[...]


def _tc_pool(hn, pred_w, pred_b):
    n, d = hn.shape
    return pl.pallas_call(
        functools.partial(_pool_body, float(n)),
        in_specs=[
            pl.BlockSpec((n, d), lambda: (0, 0)),
            pl.BlockSpec((1, d), lambda: (0, 0)),
            pl.BlockSpec((1, 1), lambda: (0, 0)),
        ],
        out_specs=pl.BlockSpec((1, 1), lambda: (0, 0)),
        out_shape=jax.ShapeDtypeStruct((1, 1), jnp.float32),
    )(hn, pred_w.reshape(1, d), pred_b.reshape(1, 1))


def _sc_aggregate(hn, he, src, dst, chunk):
    n, d = hn.shape
    e = src.shape[0]
    e_per = e // NW
    n_chunks = e_per // chunk
    n_acc = ((n + 16 + NS * 8 - 1) // (NS * 8)) * (NS * 8)
    rows_per_sub = n_acc // NS
    mesh = plsc.VectorSubcoreMesh(core_axis_name="c", subcore_axis_name="s")

    @functools.partial(
        pl.kernel,
        out_type=jax.ShapeDtypeStruct((NC, n_acc, d), jnp.float32),
        mesh=mesh,
        scratch_types=[
            pltpu.VMEM((chunk,), jnp.int32),
            pltpu.VMEM((chunk,), jnp.int32),
            pltpu.VMEM((chunk, d), jnp.float32),
            pltpu.VMEM_SHARED((n_acc, d), jnp.float32),
            pltpu.SemaphoreType.DMA,
        ],
    )
    def body(hn_hbm, he_hbm, src_hbm, dst_hbm, z_hbm, out_hbm,
             sidx, didx, rows, acc, sem):
        c = lax.axis_index("c")
        s = lax.axis_index("s")
        wid = c * NS + s
        pltpu.sync_copy(
            z_hbm.at[pl.ds(s * rows_per_sub, rows_per_sub)],
            acc.at[pl.ds(s * rows_per_sub, rows_per_sub)],
        )
        plsc.subcore_barrier()

        base0 = wid * e_per

        def chunk_body(j, carry):
            base = base0 + j * chunk
            pltpu.sync_copy(src_hbm.at[pl.ds(base, chunk)], sidx)
            pltpu.sync_copy(dst_hbm.at[pl.ds(base, chunk)], didx)
            pltpu.sync_copy(he_hbm.at[pl.ds(base, chunk)], rows)
            pltpu.async_copy(hn_hbm.at[sidx], rows, sem, add=True).wait()

            def relu_row(r, carry2):
                for kk in range(d // LANES):
                    sl = pl.ds(kk * LANES, LANES)
                    rows[r, sl] = jnp.maximum(rows[r, sl], 0.0)
                return carry2

            lax.fori_loop(0, chunk, relu_row, 0, unroll=2)
            pltpu.sync_copy(rows, acc.at[didx], add=True)
            return carry

        lax.fori_loop(0, n_chunks, chunk_body, 0)
        plsc.subcore_barrier()
        pltpu.sync_copy(
            acc.at[pl.ds(s * rows_per_sub, rows_per_sub)],
            out_hbm.at[c, pl.ds(s * rows_per_sub, rows_per_sub)],
        )

    zeros = jnp.zeros((n_acc, d), jnp.float32)
    return body(hn, he, src, dst, zeros)


def kernel(node_feat, edge_feat, edge_index, node_W, node_b, edge_W, edge_b,
           W1, b1, g1, be1, W2, b2, g2, be2, pred_W, pred_b):
    n, d = node_feat.shape
    e = edge_feat.shape[0]
    num_layers = W1.shape[0]

    chunk = 80
    e_pad = ((e + NW * chunk - 1) // (NW * chunk)) * (NW * chunk)
    src = edge_index[0]
    dst = edge_index[1]
    if e_pad != e:
        src = jnp.concatenate([src, jnp.zeros((e_pad - e,), jnp.int32)])
        dst = jnp.concatenate([dst, jnp.full((e_pad - e,), n, jnp.int32)])

    s1 = (g1 / jnp.sqrt(1.0 + 1e-5))
    s2 = (g2 / jnp.sqrt(1.0 + 1e-5))
    W1f = W1 * s1[:, None, :]
    b1f = b1 * s1 + be1
    W2f = W2 * s2[:, None, :]
    b2f = b2 * s2 + be2

    hn = _tc_linear(node_feat, node_W, node_b, block_rows=1000)
    for l in range(num_layers):
        he = _tc_linear(edge_feat, edge_W[l], edge_b[l], block_rows=3200)
        if e_pad != e:
            he = jnp.concatenate(
                [he, jnp.zeros((e_pad - e, d), jnp.float32)], axis=0)
        aggr = _sc_aggregate(hn, he, src, dst, chunk)
        aggr = aggr[:, :n]
        hn = _tc_mlp(hn, aggr, W1f[l], b1f[l], W2f[l], b2f[l],
                     relu_out=(l != num_layers - 1), block_rows=1000)

    return _tc_pool(hn, pred_W, pred_b)


# speedup vs baseline: 5.2760x; 1.9664x over previous
import functools

import jax
import jax.numpy as jnp
from jax import lax
from jax.experimental import pallas as pl
from jax.experimental.pallas import tpu as pltpu
from jax.experimental.pallas import tpu_sc as plsc

NC = 2
NS = 16
LANES = 16
NW = NC * NS


def _linear_body(x_ref, w_ref, b_ref, o_ref):
    o_ref[...] = (
        jnp.dot(x_ref[...], w_ref[...], preferred_element_type=jnp.float32)
        + b_ref[...]
    )


def _tc_linear(x, w, b, block_rows):
    m, k = x.shape
    n = w.shape[1]
    assert m % block_rows == 0
    return pl.pallas_call(
        _linear_body,
        grid=(m // block_rows,),
        in_specs=[
            pl.BlockSpec((block_rows, k), lambda i: (i, 0)),
            pl.BlockSpec((k, n), lambda i: (0, 0)),
            pl.BlockSpec((1, n), lambda i: (0, 0)),
        ],
        out_specs=pl.BlockSpec((block_rows, n), lambda i: (i, 0)),
        out_shape=jax.ShapeDtypeStruct((m, n), jnp.float32),
    )(x, w, b.reshape(1, n))


def _mlp_body(relu_out, hn_ref, a_ref, w1_ref, b1_ref, w2_ref, b2_ref, o_ref):
    h = hn_ref[...] + a_ref[0] + a_ref[1]
    y = jnp.dot(h, w1_ref[...], preferred_element_type=jnp.float32) + b1_ref[...]
    y = jnp.maximum(y, 0.0)
    z = jnp.dot(y, w2_ref[...], preferred_element_type=jnp.float32) + b2_ref[...]
    if relu_out:
        z = jnp.maximum(z, 0.0)
    o_ref[...] = z


def _tc_mlp(hn, aggr, w1, b1, w2, b2, relu_out, block_rows):
    n, d = hn.shape
    h = w1.shape[1]
    assert n % block_rows == 0
    return pl.pallas_call(
        functools.partial(_mlp_body, relu_out),
        grid=(n // block_rows,),
        in_specs=[
            pl.BlockSpec((block_rows, d), lambda i: (i, 0)),
            pl.BlockSpec((NC, block_rows, d), lambda i: (0, i, 0)),
            pl.BlockSpec((d, h), lambda i: (0, 0)),
            pl.BlockSpec((1, h), lambda i: (0, 0)),
            pl.BlockSpec((h, d), lambda i: (0, 0)),
            pl.BlockSpec((1, d), lambda i: (0, 0)),
        ],
        out_specs=pl.BlockSpec((block_rows, d), lambda i: (i, 0)),
        out_shape=jax.ShapeDtypeStruct((n, d), jnp.float32),
    )(hn, aggr, w1, b1.reshape(1, h), w2, b2.reshape(1, d))


def _pool_body(n_rows, hn_ref, wt_ref, b_ref, o_ref):
    s = jnp.sum(hn_ref[...], axis=0, keepdims=True) * (1.0 / n_rows)
    o_ref[...] = jnp.sum(s * wt_ref[...], axis=1, keepdims=True) + b_ref[...]


def _tc_pool(hn, pred_w, pred_b):
    n, d = hn.shape
    return pl.pallas_call(
        functools.partial(_pool_body, float(n)),
        in_specs=[
            pl.BlockSpec((n, d), lambda: (0, 0)),
            pl.BlockSpec((1, d), lambda: (0, 0)),
            pl.BlockSpec((1, 1), lambda: (0, 0)),
        ],
        out_specs=pl.BlockSpec((1, 1), lambda: (0, 0)),
        out_shape=jax.ShapeDtypeStruct((1, 1), jnp.float32),
    )(hn, pred_w.reshape(1, d), pred_b.reshape(1, 1))


def _sc_aggregate(hn, he, src, dst, chunk, nbuf=5):
    n, d = hn.shape
    e = src.shape[0]
    e_per = e // NW
    n_slots = e_per // chunk
    assert n_slots % nbuf == 0
    n_acc = ((n + 1 + NS * 8 - 1) // (NS * 8)) * (NS * 8)
    rows_per_sub = n_acc // NS
    pf = nbuf - 1
    ga = 2
    mesh = plsc.VectorSubcoreMesh(core_axis_name="c", subcore_axis_name="s")

    @functools.partial(
        pl.kernel,
        out_type=jax.ShapeDtypeStruct((NC, n_acc, d), jnp.float32),
        mesh=mesh,
        scratch_types=[
            pltpu.VMEM((nbuf, chunk), jnp.int32),
            pltpu.VMEM((nbuf, chunk), jnp.int32),
            pltpu.VMEM((nbuf, chunk, d), jnp.float32),
            pltpu.VMEM_SHARED((n_acc, d), jnp.float32),
            pltpu.SemaphoreType.DMA((nbuf,)),
            pltpu.SemaphoreType.DMA((nbuf,)),
        ],
    )
    def body(hn_hbm, he_hbm, src_hbm, dst_hbm, z_hbm, out_hbm,
             sidx, didx, rows, acc, sem_in, sem_g):
        c = lax.axis_index("c")
        s = lax.axis_index("s")
        wid = c * NS + s
        pltpu.sync_copy(
            z_hbm.at[pl.ds(s * rows_per_sub, rows_per_sub)],
            acc.at[pl.ds(s * rows_per_sub, rows_per_sub)],
        )

        base0 = wid * e_per

        def issue_in(slot, b):
            base = base0 + slot * chunk
            pltpu.async_copy(src_hbm.at[pl.ds(base, chunk)], sidx.at[b],
                             sem_in.at[b])
            pltpu.async_copy(dst_hbm.at[pl.ds(base, chunk)], didx.at[b],
                             sem_in.at[b])
            pltpu.async_copy(he_hbm.at[pl.ds(base, chunk)], rows.at[b],
                             sem_in.at[b])

        def wait_in(b):
            pltpu.make_async_copy(src_hbm.at[pl.ds(0, chunk)], sidx.at[b],
                                  sem_in.at[b]).wait()
            pltpu.make_async_copy(dst_hbm.at[pl.ds(0, chunk)], didx.at[b],
                                  sem_in.at[b]).wait()
            pltpu.make_async_copy(he_hbm.at[pl.ds(0, chunk)], rows.at[b],
                                  sem_in.at[b]).wait()

        def issue_gather(b):
            wait_in(b)
            pltpu.async_copy(hn_hbm.at[sidx.at[b]], rows.at[b], sem_g.at[b],
                             add=True)

        def wait_gather(b):
            pltpu.make_async_copy(hn_hbm.at[sidx.at[b]], rows.at[b],
                                  sem_g.at[b]).wait()

        plsc.subcore_barrier()

        for slot in range(min(pf, n_slots)):
            issue_in(slot, slot % nbuf)
        for slot in range(min(ga, n_slots)):
            issue_gather(slot % nbuf)

        @pl.loop(0, n_slots, step=nbuf)
        def outer(i):
            for b in range(nbuf):
                slot = i + b

                @pl.when(slot + pf < n_slots)
                def _():
                    issue_in(slot + pf, (b + pf) % nbuf)

                @pl.when(slot + ga < n_slots)
                def _():
                    issue_gather((b + ga) % nbuf)

                wait_gather(b)

                @pl.loop(0, chunk, unroll=4)
                def relu_row(r):
                    for kk in range(d // LANES):
                        sl = pl.ds(kk * LANES, LANES)
                        rows[b, r, sl] = jnp.maximum(rows[b, r, sl], 0.0)

                pltpu.sync_copy(rows.at[b], acc.at[didx.at[b]], add=True)

        plsc.subcore_barrier()
        pltpu.sync_copy(
            acc.at[pl.ds(s * rows_per_sub, rows_per_sub)],
            out_hbm.at[c, pl.ds(s * rows_per_sub, rows_per_sub)],
        )

    zeros = jnp.zeros((n_acc, d), jnp.float32)
    return body(hn, he, src, dst, zeros)


def kernel(node_feat, edge_feat, edge_index, node_W, node_b, edge_W, edge_b,
           W1, b1, g1, be1, W2, b2, g2, be2, pred_W, pred_b):
    n, d = node_feat.shape
    e = edge_feat.shape[0]
    num_layers = W1.shape[0]

    chunk = 40
    e_pad = ((e + NW * chunk - 1) // (NW * chunk)) * (NW * chunk)
    src = edge_index[0]
    dst = edge_index[1]
    if e_pad != e:
        src = jnp.concatenate([src, jnp.zeros((e_pad - e,), jnp.int32)])
        dst = jnp.concatenate([dst, jnp.full((e_pad - e,), n, jnp.int32)])

    s1 = (g1 / jnp.sqrt(1.0 + 1e-5))
    s2 = (g2 / jnp.sqrt(1.0 + 1e-5))
    W1f = W1 * s1[:, None, :]
    b1f = b1 * s1 + be1
    W2f = W2 * s2[:, None, :]
    b2f = b2 * s2 + be2

    hn = _tc_linear(node_feat, node_W, node_b, block_rows=1000)
    for l in range(num_layers):
        he = _tc_linear(edge_feat, edge_W[l], edge_b[l], block_rows=3200)
        if e_pad != e:
            he = jnp.concatenate(
                [he, jnp.zeros((e_pad - e, d), jnp.float32)], axis=0)
        aggr = _sc_aggregate(hn, he, src, dst, chunk)
        aggr = aggr[:, :n]
        hn = _tc_mlp(hn, aggr, W1f[l], b1f[l], W2f[l], b2f[l],
                     relu_out=(l != num_layers - 1), block_rows=1000)

    return _tc_pool(hn, pred_W, pred_b)
